# Initial kernel scaffold; baseline (speedup 1.0000x reference)
import jax
import jax.numpy as jnp
from jax.experimental import pallas as pl


def kernel(x, edge_index, batch, adme_features, sage_Wself, sage_Wneigh, sage_b, hW0, hb0, g0, b0, hW1, hb1, g1, b1, hW2, hb2, g2, b2, hW3, hb3):
    raise NotImplementedError("write your pallas kernel here")


# speedup vs baseline: 2.4376x; 2.4376x over previous
import functools

import jax
import jax.numpy as jnp
from jax import lax
from jax.experimental import pallas as pl
from jax.experimental.pallas import tpu as pltpu
from jax.experimental.pallas import tpu_sc as plsc

_N = 10000
_E = 320000
_D = 128
_B = 64
_L = 5
_EPS = 1e-5

_NC = 2
_NS = 16
_TILES = _NC * _NS
_CHUNK = 128
_CPT = 80
_TOT_CHUNKS = _CPT * _TILES
_E_PAD = _TOT_CHUNKS * _CHUNK
_NPAD = 10112
_RPT = _NPAD // _NS
_IDXG = 8


def _make_sc_agg():
    mesh = plsc.VectorSubcoreMesh(core_axis_name="c", subcore_axis_name="s",
                                  num_cores=_NC, num_subcores=_NS)
    out_type = [jax.ShapeDtypeStruct((_NC * _NPAD, _D), jnp.float32)]
    scratch = [
        pltpu.VMEM((_IDXG, _CHUNK), jnp.int32),
        pltpu.VMEM((_IDXG, _CHUNK), jnp.int32),
        pltpu.VMEM((_CHUNK, _D), jnp.float32),
        pltpu.VMEM_SHARED((_NPAD, _D), jnp.float32),
        pltpu.SemaphoreType.DMA,
    ]

    _ZCHUNKS = (128, 128, 128, 128, _RPT - 512)

    def body(h_hbm, src_hbm, dst_hbm, agg_out, src_v, dst_v, rows_v, agg_sh, sem):
        c = lax.axis_index("c")
        s = lax.axis_index("s")
        wid = c * _NS + s
        zero16 = jnp.zeros((16,), jnp.float32)

        def zero_rows(r, carry):
            for kk in range(_D // 16):
                rows_v[r, pl.ds(kk * 16, 16)] = zero16
            return carry
        lax.fori_loop(0, _CHUNK, zero_rows, 0)

        base = s * _RPT
        off = 0
        for nn in _ZCHUNKS:
            pltpu.sync_copy(rows_v.at[pl.ds(0, nn)], agg_sh.at[pl.ds(base + off, nn)])
            off += nn

        plsc.subcore_barrier()

        def group(g, carry):
            gbase = wid * _CPT + g * _IDXG
            pltpu.sync_copy(src_hbm.at[pl.ds(gbase, _IDXG)], src_v)
            pltpu.sync_copy(dst_hbm.at[pl.ds(gbase, _IDXG)], dst_v)

            def step(j, carry2):
                pltpu.async_copy(h_hbm.at[src_v.at[j]], rows_v, sem).wait()
                pltpu.sync_copy(rows_v, agg_sh.at[dst_v.at[j]], add=True)
                return carry2
            lax.fori_loop(0, _IDXG, step, 0)
            return carry
        lax.fori_loop(0, _CPT // _IDXG, group, 0)

        plsc.subcore_barrier()

        obase = c * _NPAD + base
        pltpu.sync_copy(agg_sh.at[pl.ds(base, _RPT)], agg_out.at[pl.ds(obase, _RPT)])

    return pl.kernel(body, out_type=out_type, mesh=mesh, scratch_types=scratch)


_RL = 1024
_GL = 10

def _tc_layer(h, a0, a1, d0, d1, wsT, wnT, b):
    def body(h_r, a0_r, a1_r, d0_r, d1_r, ws_r, wn_r, b_r, o_r):
        deg = jnp.maximum(d0_r[:, 0:1] + d1_r[:, 0:1], 1.0)
        mean = (a0_r[...] + a1_r[...]) / deg
        acc = jnp.dot(h_r[...], ws_r[...], preferred_element_type=jnp.float32)
        acc = acc + jnp.dot(mean, wn_r[...], preferred_element_type=jnp.float32)
        o_r[...] = jnp.maximum(acc + b_r[...], 0.0)

    bs_rows = pl.BlockSpec((_RL, _D), lambda i: (i, 0))
    bs_d = pl.BlockSpec((_RL, 16), lambda i: (i, 0))
    bs_w = pl.BlockSpec((_D, _D), lambda i: (0, 0))
    bs_b = pl.BlockSpec((1, _D), lambda i: (0, 0))
    return pl.pallas_call(
        body,
        grid=(_GL,),
        in_specs=[bs_rows, bs_rows, bs_rows, bs_d, bs_d, bs_w, bs_w, bs_b],
        out_specs=bs_rows,
        out_shape=jax.ShapeDtypeStruct((_N, _D), jnp.float32),
    )(h, a0, a1, d0, d1, wsT, wnT, b)


_RH = 1024
_GH = 10
_NH = _RH * _GH


def _bn_relu(z, g, b):
    m = jnp.mean(z, axis=0, keepdims=True)
    zc = z - m
    v = jnp.mean(zc * zc, axis=0, keepdims=True)
    return jnp.maximum(zc / jnp.sqrt(v + _EPS) * g + b, 0.0)


def _tc_head(hp, bt3, adme_p, w0p, w0a, hb0, g0, b0, w1t, hb1, g1, b1,
             w2t, hb2, g2, b2, w3t, hb3):
    def body(h_r, bt_r, adme_r, w0p_r, w0a_r, hb0_r, g0_r, b0_r,
             w1_r, hb1_r, g1_r, b1_r, w2_r, hb2_r, g2_r, b2_r,
             w3_r, hb3_r, o_r, pool_acc, cnt_acc):
        i = pl.program_id(0)

        @pl.when(i == 0)
        def _():
            pool_acc[...] = jnp.zeros_like(pool_acc)
            cnt_acc[...] = jnp.zeros_like(cnt_acc)

        bt = bt_r[...].reshape(1, _RH)
        ids = lax.broadcasted_iota(jnp.int32, (_B, _RH), 0)
        mask = (bt == ids).astype(jnp.float32)
        pool_acc[...] += jnp.dot(mask, h_r[...], preferred_element_type=jnp.float32)
        cnt_acc[...] += jnp.broadcast_to(
            jnp.sum(mask, axis=1, keepdims=True), (_B, _D))

        @pl.when(i == _GH - 1)
        def _():
            pooled = pool_acc[...] / jnp.maximum(cnt_acc[...], 1.0)
            z = jnp.dot(pooled, w0p_r[...], preferred_element_type=jnp.float32)
            z = z + jnp.dot(adme_r[...], w0a_r[...], preferred_element_type=jnp.float32)
            z = _bn_relu(z + hb0_r[...], g0_r[...], b0_r[...])
            z = jnp.dot(z, w1_r[...], preferred_element_type=jnp.float32) + hb1_r[...]
            z = _bn_relu(z, g1_r[...], b1_r[...])
            z = jnp.dot(z, w2_r[...], preferred_element_type=jnp.float32) + hb2_r[...]
            z = _bn_relu(z, g2_r[...], b2_r[...])
            o_r[...] = jnp.dot(z, w3_r[...], preferred_element_type=jnp.float32) + hb3_r[...]

    def full(shape):
        return pl.BlockSpec(shape, lambda i: tuple(0 for _ in shape))

    return pl.pallas_call(
        body,
        grid=(_GH,),
        in_specs=[
            pl.BlockSpec((_RH, _D), lambda i: (i, 0)),
            pl.BlockSpec((1, 1, _RH), lambda i: (i, 0, 0)),
            full((_B, 16)),
            full((_D, 256)), full((16, 256)), full((1, 256)), full((1, 256)), full((1, 256)),
            full((256, _D)), full((1, _D)), full((1, _D)), full((1, _D)),
            full((_D, 64)), full((1, 64)), full((1, 64)), full((1, 64)),
            full((64, 1)), full((1, 1)),
        ],
        out_specs=pl.BlockSpec((_B, 1), lambda i: (0, 0)),
        out_shape=jax.ShapeDtypeStruct((_B, 1), jnp.float32),
        scratch_shapes=[
            pltpu.VMEM((_B, _D), jnp.float32),
            pltpu.VMEM((_B, _D), jnp.float32),
        ],
    )(hp, bt3, adme_p, w0p, w0a, hb0, g0, b0, w1t, hb1, g1, b1,
      w2t, hb2, g2, b2, w3t, hb3)


def kernel(x, edge_index, batch, adme_features, sage_Wself, sage_Wneigh, sage_b,
           hW0, hb0, g0, b0, hW1, hb1, g1, b1, hW2, hb2, g2, b2, hW3, hb3):
    pad = _E_PAD - _E
    src_p = jnp.concatenate(
        [edge_index[0], jnp.zeros((pad,), jnp.int32)]).reshape(_TOT_CHUNKS, _CHUNK)
    dst_p = jnp.concatenate(
        [edge_index[1], jnp.full((pad,), _N, jnp.int32)]).reshape(_TOT_CHUNKS, _CHUNK)

    agg_fn = _make_sc_agg()

    def run_agg(tab):
        r = agg_fn(tab, src_p, dst_p)
        if isinstance(r, (list, tuple)):
            r = r[0]
        return r

    deg_flat = run_agg(jnp.ones((_N, _D), jnp.float32))
    d0 = deg_flat[:_N, :16]
    d1 = deg_flat[_NPAD:_NPAD + _N, :16]

    h = x
    for i in range(_L):
        agg_flat = run_agg(h)
        a0 = agg_flat[:_N]
        a1 = agg_flat[_NPAD:_NPAD + _N]
        h = _tc_layer(h, a0, a1, d0, d1, sage_Wself[i].T, sage_Wneigh[i].T,
                      sage_b[i].reshape(1, _D))

    hp = jnp.concatenate([h, jnp.zeros((_NH - _N, _D), jnp.float32)], axis=0)
    bt3 = jnp.concatenate(
        [batch, jnp.full((_NH - _N,), _B, jnp.int32)]).reshape(_GH, 1, _RH)
    adme_p = jnp.pad(adme_features, ((0, 0), (0, 1)))
    w0p = hW0[:, :_D].T
    w0a = jnp.pad(hW0[:, _D:].T, ((0, 1), (0, 0)))

    out = _tc_head(hp, bt3, adme_p, w0p, w0a,
                   hb0.reshape(1, -1), g0.reshape(1, -1), b0.reshape(1, -1),
                   hW1.T, hb1.reshape(1, -1), g1.reshape(1, -1), b1.reshape(1, -1),
                   hW2.T, hb2.reshape(1, -1), g2.reshape(1, -1), b2.reshape(1, -1),
                   hW3.T, hb3.reshape(1, 1))
    return out.reshape(_B)


# speedup vs baseline: 3.0718x; 1.2602x over previous
import functools

import jax
import jax.numpy as jnp
from jax import lax
from jax.experimental import pallas as pl
from jax.experimental.pallas import tpu as pltpu
from jax.experimental.pallas import tpu_sc as plsc

_N = 10000
_E = 320000
_D = 128
_B = 64
_L = 5
_EPS = 1e-5

_NC = 2
_NS = 16
_TILES = _NC * _NS
_CHUNK = 128
_CPT = 80
_TOT_CHUNKS = _CPT * _TILES
_E_PAD = _TOT_CHUNKS * _CHUNK
_NPAD = 10112
_RPT = _NPAD // _NS
_IDXG = 16


def _make_sc_agg():
    mesh = plsc.VectorSubcoreMesh(core_axis_name="c", subcore_axis_name="s",
                                  num_cores=_NC, num_subcores=_NS)
    out_type = [jax.ShapeDtypeStruct((_NC * _NPAD, _D), jnp.float32)]
    scratch = [
        pltpu.VMEM((_IDXG, _CHUNK), jnp.int32),
        pltpu.VMEM((_IDXG, _CHUNK), jnp.int32),
        pltpu.VMEM((_CHUNK, _D), jnp.float32),
        pltpu.VMEM((_CHUNK, _D), jnp.float32),
        pltpu.VMEM_SHARED((_NPAD, _D), jnp.float32),
        pltpu.SemaphoreType.DMA,
        pltpu.SemaphoreType.DMA,
    ]

    _ZCHUNKS = (128, 128, 128, 128, _RPT - 512)

    def body(h_hbm, src_hbm, dst_hbm, agg_out, src_v, dst_v, rows_a, rows_b,
             agg_sh, sem_a, sem_b):
        c = lax.axis_index("c")
        s = lax.axis_index("s")
        wid = c * _NS + s
        zero16 = jnp.zeros((16,), jnp.float32)

        def zero_rows(r, carry):
            for kk in range(_D // 16):
                rows_a[r, pl.ds(kk * 16, 16)] = zero16
            return carry
        lax.fori_loop(0, _CHUNK, zero_rows, 0)

        base = s * _RPT
        off = 0
        for nn in _ZCHUNKS:
            pltpu.sync_copy(rows_a.at[pl.ds(0, nn)], agg_sh.at[pl.ds(base + off, nn)])
            off += nn

        plsc.subcore_barrier()

        bufs = (rows_a, rows_b)
        sems = (sem_a, sem_b)

        def group(g, carry):
            gbase = wid * _CPT + g * _IDXG
            pltpu.sync_copy(src_hbm.at[pl.ds(gbase, _IDXG)], src_v)
            pltpu.sync_copy(dst_hbm.at[pl.ds(gbase, _IDXG)], dst_v)
            descs = {0: pltpu.async_copy(h_hbm.at[src_v.at[0]], bufs[0], sems[0])}
            for j in range(_IDXG):
                if j + 1 < _IDXG:
                    descs[j + 1] = pltpu.async_copy(
                        h_hbm.at[src_v.at[j + 1]], bufs[(j + 1) % 2], sems[(j + 1) % 2])
                descs[j].wait()
                pltpu.sync_copy(bufs[j % 2], agg_sh.at[dst_v.at[j]], add=True)
            return carry
        lax.fori_loop(0, _CPT // _IDXG, group, 0)

        plsc.subcore_barrier()

        obase = c * _NPAD + base
        pltpu.sync_copy(agg_sh.at[pl.ds(base, _RPT)], agg_out.at[pl.ds(obase, _RPT)])

    return pl.kernel(body, out_type=out_type, mesh=mesh, scratch_types=scratch)


_RL = 1024
_GL = 10

def _tc_layer(h, a0, a1, d0, d1, wsT, wnT, b):
    def body(h_r, a0_r, a1_r, d0_r, d1_r, ws_r, wn_r, b_r, o_r):
        deg = jnp.maximum(d0_r[:, 0:1] + d1_r[:, 0:1], 1.0)
        mean = (a0_r[...] + a1_r[...]) / deg
        acc = jnp.dot(h_r[...], ws_r[...], preferred_element_type=jnp.float32)
        acc = acc + jnp.dot(mean, wn_r[...], preferred_element_type=jnp.float32)
        o_r[...] = jnp.maximum(acc + b_r[...], 0.0)

    bs_rows = pl.BlockSpec((_RL, _D), lambda i: (i, 0))
    bs_d = pl.BlockSpec((_RL, 16), lambda i: (i, 0))
    bs_w = pl.BlockSpec((_D, _D), lambda i: (0, 0))
    bs_b = pl.BlockSpec((1, _D), lambda i: (0, 0))
    return pl.pallas_call(
        body,
        grid=(_GL,),
        in_specs=[bs_rows, bs_rows, bs_rows, bs_d, bs_d, bs_w, bs_w, bs_b],
        out_specs=bs_rows,
        out_shape=jax.ShapeDtypeStruct((_N, _D), jnp.float32),
    )(h, a0, a1, d0, d1, wsT, wnT, b)


_RH = 1024
_GH = 10
_NH = _RH * _GH


def _bn_relu(z, g, b):
    m = jnp.mean(z, axis=0, keepdims=True)
    zc = z - m
    v = jnp.mean(zc * zc, axis=0, keepdims=True)
    return jnp.maximum(zc / jnp.sqrt(v + _EPS) * g + b, 0.0)


def _tc_head(hp, bt3, adme_p, w0p, w0a, hb0, g0, b0, w1t, hb1, g1, b1,
             w2t, hb2, g2, b2, w3t, hb3):
    def body(h_r, bt_r, adme_r, w0p_r, w0a_r, hb0_r, g0_r, b0_r,
             w1_r, hb1_r, g1_r, b1_r, w2_r, hb2_r, g2_r, b2_r,
             w3_r, hb3_r, o_r, pool_acc, cnt_acc):
        i = pl.program_id(0)

        @pl.when(i == 0)
        def _():
            pool_acc[...] = jnp.zeros_like(pool_acc)
            cnt_acc[...] = jnp.zeros_like(cnt_acc)

        bt = bt_r[...].reshape(1, _RH)
        ids = lax.broadcasted_iota(jnp.int32, (_B, _RH), 0)
        mask = (bt == ids).astype(jnp.float32)
        pool_acc[...] += jnp.dot(mask, h_r[...], preferred_element_type=jnp.float32)
        cnt_acc[...] += jnp.broadcast_to(
            jnp.sum(mask, axis=1, keepdims=True), (_B, _D))

        @pl.when(i == _GH - 1)
        def _():
            pooled = pool_acc[...] / jnp.maximum(cnt_acc[...], 1.0)
            z = jnp.dot(pooled, w0p_r[...], preferred_element_type=jnp.float32)
            z = z + jnp.dot(adme_r[...], w0a_r[...], preferred_element_type=jnp.float32)
            z = _bn_relu(z + hb0_r[...], g0_r[...], b0_r[...])
            z = jnp.dot(z, w1_r[...], preferred_element_type=jnp.float32) + hb1_r[...]
            z = _bn_relu(z, g1_r[...], b1_r[...])
            z = jnp.dot(z, w2_r[...], preferred_element_type=jnp.float32) + hb2_r[...]
            z = _bn_relu(z, g2_r[...], b2_r[...])
            o_r[...] = jnp.dot(z, w3_r[...], preferred_element_type=jnp.float32) + hb3_r[...]

    def full(shape):
        return pl.BlockSpec(shape, lambda i: tuple(0 for _ in shape))

    return pl.pallas_call(
        body,
        grid=(_GH,),
        in_specs=[
            pl.BlockSpec((_RH, _D), lambda i: (i, 0)),
            pl.BlockSpec((1, 1, _RH), lambda i: (i, 0, 0)),
            full((_B, 16)),
            full((_D, 256)), full((16, 256)), full((1, 256)), full((1, 256)), full((1, 256)),
            full((256, _D)), full((1, _D)), full((1, _D)), full((1, _D)),
            full((_D, 64)), full((1, 64)), full((1, 64)), full((1, 64)),
            full((64, 1)), full((1, 1)),
        ],
        out_specs=pl.BlockSpec((_B, 1), lambda i: (0, 0)),
        out_shape=jax.ShapeDtypeStruct((_B, 1), jnp.float32),
        scratch_shapes=[
            pltpu.VMEM((_B, _D), jnp.float32),
            pltpu.VMEM((_B, _D), jnp.float32),
        ],
    )(hp, bt3, adme_p, w0p, w0a, hb0, g0, b0, w1t, hb1, g1, b1,
      w2t, hb2, g2, b2, w3t, hb3)


def kernel(x, edge_index, batch, adme_features, sage_Wself, sage_Wneigh, sage_b,
           hW0, hb0, g0, b0, hW1, hb1, g1, b1, hW2, hb2, g2, b2, hW3, hb3):
    pad = _E_PAD - _E
    src_p = jnp.concatenate(
        [edge_index[0], jnp.zeros((pad,), jnp.int32)]).reshape(_TOT_CHUNKS, _CHUNK)
    dst_p = jnp.concatenate(
        [edge_index[1], jnp.full((pad,), _N, jnp.int32)]).reshape(_TOT_CHUNKS, _CHUNK)

    agg_fn = _make_sc_agg()

    def run_agg(tab):
        r = agg_fn(tab, src_p, dst_p)
        if isinstance(r, (list, tuple)):
            r = r[0]
        return r

    deg_flat = run_agg(jnp.ones((_N, _D), jnp.float32))
    d0 = deg_flat[:_N, :16]
    d1 = deg_flat[_NPAD:_NPAD + _N, :16]

    h = x
    for i in range(_L):
        agg_flat = run_agg(h)
        a0 = agg_flat[:_N]
        a1 = agg_flat[_NPAD:_NPAD + _N]
        h = _tc_layer(h, a0, a1, d0, d1, sage_Wself[i].T, sage_Wneigh[i].T,
                      sage_b[i].reshape(1, _D))

    hp = jnp.concatenate([h, jnp.zeros((_NH - _N, _D), jnp.float32)], axis=0)
    bt3 = jnp.concatenate(
        [batch, jnp.full((_NH - _N,), _B, jnp.int32)]).reshape(_GH, 1, _RH)
    adme_p = jnp.pad(adme_features, ((0, 0), (0, 1)))
    w0p = hW0[:, :_D].T
    w0a = jnp.pad(hW0[:, _D:].T, ((0, 1), (0, 0)))

    out = _tc_head(hp, bt3, adme_p, w0p, w0a,
                   hb0.reshape(1, -1), g0.reshape(1, -1), b0.reshape(1, -1),
                   hW1.T, hb1.reshape(1, -1), g1.reshape(1, -1), b1.reshape(1, -1),
                   hW2.T, hb2.reshape(1, -1), g2.reshape(1, -1), b2.reshape(1, -1),
                   hW3.T, hb3.reshape(1, 1))
    return out.reshape(_B)


# speedup vs baseline: 3.0747x; 1.0009x over previous
import functools

import jax
import jax.numpy as jnp
from jax import lax
from jax.experimental import pallas as pl
from jax.experimental.pallas import tpu as pltpu
from jax.experimental.pallas import tpu_sc as plsc

_N = 10000
_E = 320000
_D = 128
_B = 64
_L = 5
_EPS = 1e-5

_NC = 2
_NS = 16
_TILES = _NC * _NS
_CHUNK = 128
_CPT = 80
_TOT_CHUNKS = _CPT * _TILES
_E_PAD = _TOT_CHUNKS * _CHUNK
_NPAD = 10112
_RPT = _NPAD // _NS
_IDXG = 16


def _make_sc_deg():
    mesh = plsc.VectorSubcoreMesh(core_axis_name="c", subcore_axis_name="s",
                                  num_cores=_NC, num_subcores=_NS)
    out_type = [jax.ShapeDtypeStruct((_NC * _NPAD, _D), jnp.float32)]
    scratch = [
        pltpu.VMEM((_IDXG, _CHUNK), jnp.int32),
        pltpu.VMEM((_CHUNK, _D), jnp.float32),
        pltpu.VMEM_SHARED((_NPAD, _D), jnp.float32),
    ]

    _ZCHUNKS = (128, 128, 128, 128, _RPT - 512)

    def body(dst_hbm, deg_out, dst_v, rows_v, agg_sh):
        c = lax.axis_index("c")
        s = lax.axis_index("s")
        wid = c * _NS + s
        zero16 = jnp.zeros((16,), jnp.float32)

        def zero_rows(r, carry):
            for kk in range(_D // 16):
                rows_v[r, pl.ds(kk * 16, 16)] = zero16
            return carry
        lax.fori_loop(0, _CHUNK, zero_rows, 0)

        base = s * _RPT
        off = 0
        for nn in _ZCHUNKS:
            pltpu.sync_copy(rows_v.at[pl.ds(0, nn)], agg_sh.at[pl.ds(base + off, nn)])
            off += nn

        one16 = jnp.ones((16,), jnp.float32)

        def fill_rows(r, carry):
            rows_v[r, pl.ds(0, 16)] = one16
            return carry
        lax.fori_loop(0, _CHUNK, fill_rows, 0)

        plsc.subcore_barrier()

        def group(g, carry):
            gbase = wid * _CPT + g * _IDXG
            pltpu.sync_copy(dst_hbm.at[pl.ds(gbase, _IDXG)], dst_v)
            for j in range(_IDXG):
                pltpu.sync_copy(rows_v, agg_sh.at[dst_v.at[j]], add=True)
            return carry
        lax.fori_loop(0, _CPT // _IDXG, group, 0)

        plsc.subcore_barrier()

        obase = c * _NPAD + base
        pltpu.sync_copy(agg_sh.at[pl.ds(base, _RPT)], deg_out.at[pl.ds(obase, _RPT)])

    return pl.kernel(body, out_type=out_type, mesh=mesh, scratch_types=scratch)


def _make_sc_agg():
    mesh = plsc.VectorSubcoreMesh(core_axis_name="c", subcore_axis_name="s",
                                  num_cores=_NC, num_subcores=_NS)
    out_type = [jax.ShapeDtypeStruct((_NC * _NPAD, _D), jnp.float32)]
    scratch = [
        pltpu.VMEM((_IDXG, _CHUNK), jnp.int32),
        pltpu.VMEM((_IDXG, _CHUNK), jnp.int32),
        pltpu.VMEM((_CHUNK, _D), jnp.float32),
        pltpu.VMEM((_CHUNK, _D), jnp.float32),
        pltpu.VMEM_SHARED((_NPAD, _D), jnp.float32),
        pltpu.SemaphoreType.DMA,
        pltpu.SemaphoreType.DMA,
        pltpu.SemaphoreType.DMA,
        pltpu.SemaphoreType.DMA,
    ]

    _ZCHUNKS = (128, 128, 128, 128, _RPT - 512)

    def body(h_hbm, src_hbm, dst_hbm, agg_out, src_v, dst_v, rows_a, rows_b,
             agg_sh, sem_a, sem_b, ssem_a, ssem_b):
        c = lax.axis_index("c")
        s = lax.axis_index("s")
        wid = c * _NS + s
        zero16 = jnp.zeros((16,), jnp.float32)

        def zero_rows(r, carry):
            for kk in range(_D // 16):
                rows_a[r, pl.ds(kk * 16, 16)] = zero16
            return carry
        lax.fori_loop(0, _CHUNK, zero_rows, 0)

        base = s * _RPT
        off = 0
        for nn in _ZCHUNKS:
            pltpu.sync_copy(rows_a.at[pl.ds(0, nn)], agg_sh.at[pl.ds(base + off, nn)])
            off += nn

        plsc.subcore_barrier()

        bufs = (rows_a, rows_b)
        sems = (sem_a, sem_b)
        ssems = (ssem_a, ssem_b)

        def group(g, carry):
            gbase = wid * _CPT + g * _IDXG
            pltpu.sync_copy(src_hbm.at[pl.ds(gbase, _IDXG)], src_v)
            pltpu.sync_copy(dst_hbm.at[pl.ds(gbase, _IDXG)], dst_v)
            gd = {0: pltpu.async_copy(h_hbm.at[src_v.at[0]], bufs[0], sems[0])}
            sd = {}
            for j in range(_IDXG):
                if j + 1 < _IDXG:
                    if j >= 1:
                        sd[j - 1].wait()
                    gd[j + 1] = pltpu.async_copy(
                        h_hbm.at[src_v.at[j + 1]], bufs[(j + 1) % 2], sems[(j + 1) % 2])
                gd[j].wait()
                sd[j] = pltpu.async_copy(
                    bufs[j % 2], agg_sh.at[dst_v.at[j]], ssems[j % 2], add=True)
            sd[_IDXG - 2].wait()
            sd[_IDXG - 1].wait()
            return carry
        lax.fori_loop(0, _CPT // _IDXG, group, 0)

        plsc.subcore_barrier()

        obase = c * _NPAD + base
        pltpu.sync_copy(agg_sh.at[pl.ds(base, _RPT)], agg_out.at[pl.ds(obase, _RPT)])

    return pl.kernel(body, out_type=out_type, mesh=mesh, scratch_types=scratch)


_RL = 1024
_GL = 10

def _tc_layer(h, a0, a1, d0, d1, wsT, wnT, b):
    def body(h_r, a0_r, a1_r, d0_r, d1_r, ws_r, wn_r, b_r, o_r):
        deg = jnp.maximum(d0_r[:, 0:1] + d1_r[:, 0:1], 1.0)
        mean = (a0_r[...] + a1_r[...]) / deg
        acc = jnp.dot(h_r[...], ws_r[...], preferred_element_type=jnp.float32)
        acc = acc + jnp.dot(mean, wn_r[...], preferred_element_type=jnp.float32)
        o_r[...] = jnp.maximum(acc + b_r[...], 0.0)

    bs_rows = pl.BlockSpec((_RL, _D), lambda i: (i, 0))
    bs_d = pl.BlockSpec((_RL, 16), lambda i: (i, 0))
    bs_w = pl.BlockSpec((_D, _D), lambda i: (0, 0))
    bs_b = pl.BlockSpec((1, _D), lambda i: (0, 0))
    return pl.pallas_call(
        body,
        grid=(_GL,),
        in_specs=[bs_rows, bs_rows, bs_rows, bs_d, bs_d, bs_w, bs_w, bs_b],
        out_specs=bs_rows,
        out_shape=jax.ShapeDtypeStruct((_N, _D), jnp.float32),
    )(h, a0, a1, d0, d1, wsT, wnT, b)


_RH = 1024
_GH = 10
_NH = _RH * _GH


def _bn_relu(z, g, b):
    m = jnp.mean(z, axis=0, keepdims=True)
    zc = z - m
    v = jnp.mean(zc * zc, axis=0, keepdims=True)
    return jnp.maximum(zc / jnp.sqrt(v + _EPS) * g + b, 0.0)


def _tc_head(hp, bt3, adme_p, w0p, w0a, hb0, g0, b0, w1t, hb1, g1, b1,
             w2t, hb2, g2, b2, w3t, hb3):
    def body(h_r, bt_r, adme_r, w0p_r, w0a_r, hb0_r, g0_r, b0_r,
             w1_r, hb1_r, g1_r, b1_r, w2_r, hb2_r, g2_r, b2_r,
             w3_r, hb3_r, o_r, pool_acc, cnt_acc):
        i = pl.program_id(0)

        @pl.when(i == 0)
        def _():
            pool_acc[...] = jnp.zeros_like(pool_acc)
            cnt_acc[...] = jnp.zeros_like(cnt_acc)

        bt = bt_r[...].reshape(1, _RH)
        ids = lax.broadcasted_iota(jnp.int32, (_B, _RH), 0)
        mask = (bt == ids).astype(jnp.float32)
        pool_acc[...] += jnp.dot(mask, h_r[...], preferred_element_type=jnp.float32)
        cnt_acc[...] += jnp.broadcast_to(
            jnp.sum(mask, axis=1, keepdims=True), (_B, _D))

        @pl.when(i == _GH - 1)
        def _():
            pooled = pool_acc[...] / jnp.maximum(cnt_acc[...], 1.0)
            z = jnp.dot(pooled, w0p_r[...], preferred_element_type=jnp.float32)
            z = z + jnp.dot(adme_r[...], w0a_r[...], preferred_element_type=jnp.float32)
            z = _bn_relu(z + hb0_r[...], g0_r[...], b0_r[...])
            z = jnp.dot(z, w1_r[...], preferred_element_type=jnp.float32) + hb1_r[...]
            z = _bn_relu(z, g1_r[...], b1_r[...])
            z = jnp.dot(z, w2_r[...], preferred_element_type=jnp.float32) + hb2_r[...]
            z = _bn_relu(z, g2_r[...], b2_r[...])
            o_r[...] = jnp.dot(z, w3_r[...], preferred_element_type=jnp.float32) + hb3_r[...]

    def full(shape):
        return pl.BlockSpec(shape, lambda i: tuple(0 for _ in shape))

    return pl.pallas_call(
        body,
        grid=(_GH,),
        in_specs=[
            pl.BlockSpec((_RH, _D), lambda i: (i, 0)),
            pl.BlockSpec((1, 1, _RH), lambda i: (i, 0, 0)),
            full((_B, 16)),
            full((_D, 256)), full((16, 256)), full((1, 256)), full((1, 256)), full((1, 256)),
            full((256, _D)), full((1, _D)), full((1, _D)), full((1, _D)),
            full((_D, 64)), full((1, 64)), full((1, 64)), full((1, 64)),
            full((64, 1)), full((1, 1)),
        ],
        out_specs=pl.BlockSpec((_B, 1), lambda i: (0, 0)),
        out_shape=jax.ShapeDtypeStruct((_B, 1), jnp.float32),
        scratch_shapes=[
            pltpu.VMEM((_B, _D), jnp.float32),
            pltpu.VMEM((_B, _D), jnp.float32),
        ],
    )(hp, bt3, adme_p, w0p, w0a, hb0, g0, b0, w1t, hb1, g1, b1,
      w2t, hb2, g2, b2, w3t, hb3)


def kernel(x, edge_index, batch, adme_features, sage_Wself, sage_Wneigh, sage_b,
           hW0, hb0, g0, b0, hW1, hb1, g1, b1, hW2, hb2, g2, b2, hW3, hb3):
    pad = _E_PAD - _E
    src_p = jnp.concatenate(
        [edge_index[0], jnp.zeros((pad,), jnp.int32)]).reshape(_TOT_CHUNKS, _CHUNK)
    dst_p = jnp.concatenate(
        [edge_index[1], jnp.full((pad,), _N, jnp.int32)]).reshape(_TOT_CHUNKS, _CHUNK)

    agg_fn = _make_sc_agg()

    def run_agg(tab):
        r = agg_fn(tab, src_p, dst_p)
        if isinstance(r, (list, tuple)):
            r = r[0]
        return r

    deg_flat = _make_sc_deg()(dst_p)
    if isinstance(deg_flat, (list, tuple)):
        deg_flat = deg_flat[0]
    d0 = deg_flat[:_N, :16]
    d1 = deg_flat[_NPAD:_NPAD + _N, :16]

    h = x
    for i in range(_L):
        agg_flat = run_agg(h)
        a0 = agg_flat[:_N]
        a1 = agg_flat[_NPAD:_NPAD + _N]
        h = _tc_layer(h, a0, a1, d0, d1, sage_Wself[i].T, sage_Wneigh[i].T,
                      sage_b[i].reshape(1, _D))

    hp = jnp.concatenate([h, jnp.zeros((_NH - _N, _D), jnp.float32)], axis=0)
    bt3 = jnp.concatenate(
        [batch, jnp.full((_NH - _N,), _B, jnp.int32)]).reshape(_GH, 1, _RH)
    adme_p = jnp.pad(adme_features, ((0, 0), (0, 1)))
    w0p = hW0[:, :_D].T
    w0a = jnp.pad(hW0[:, _D:].T, ((0, 1), (0, 0)))

    out = _tc_head(hp, bt3, adme_p, w0p, w0a,
                   hb0.reshape(1, -1), g0.reshape(1, -1), b0.reshape(1, -1),
                   hW1.T, hb1.reshape(1, -1), g1.reshape(1, -1), b1.reshape(1, -1),
                   hW2.T, hb2.reshape(1, -1), g2.reshape(1, -1), b2.reshape(1, -1),
                   hW3.T, hb3.reshape(1, 1))
    return out.reshape(_B)


# speedup vs baseline: 3.0751x; 1.0001x over previous
import functools

import jax
import jax.numpy as jnp
from jax import lax
from jax.experimental import pallas as pl
from jax.experimental.pallas import tpu as pltpu
from jax.experimental.pallas import tpu_sc as plsc

_N = 10000
_E = 320000
_D = 128
_B = 64
_L = 5
_EPS = 1e-5

_NC = 2
_NS = 16
_TILES = _NC * _NS
_CHUNK = 128
_CPT = 80
_TOT_CHUNKS = _CPT * _TILES
_E_PAD = _TOT_CHUNKS * _CHUNK
_NPAD = 10112
_RPT = _NPAD // _NS
_IDXG = 16


def _make_sc_deg():
    mesh = plsc.VectorSubcoreMesh(core_axis_name="c", subcore_axis_name="s",
                                  num_cores=_NC, num_subcores=_NS)
    out_type = [jax.ShapeDtypeStruct((_NC * _NPAD, _D), jnp.float32)]
    scratch = [
        pltpu.VMEM((_IDXG, _CHUNK), jnp.int32),
        pltpu.VMEM((_CHUNK, _D), jnp.float32),
        pltpu.VMEM_SHARED((_NPAD, _D), jnp.float32),
    ] + [pltpu.SemaphoreType.DMA] * 8

    _ZCHUNKS = (128, 128, 128, 128, _RPT - 512)

    def body(dst_hbm, deg_out, dst_v, rows_v, agg_sh, *dsems):
        c = lax.axis_index("c")
        s = lax.axis_index("s")
        wid = c * _NS + s
        zero16 = jnp.zeros((16,), jnp.float32)

        def zero_rows(r, carry):
            for kk in range(_D // 16):
                rows_v[r, pl.ds(kk * 16, 16)] = zero16
            return carry
        lax.fori_loop(0, _CHUNK, zero_rows, 0)

        base = s * _RPT
        off = 0
        for nn in _ZCHUNKS:
            pltpu.sync_copy(rows_v.at[pl.ds(0, nn)], agg_sh.at[pl.ds(base + off, nn)])
            off += nn

        one16 = jnp.ones((16,), jnp.float32)

        def fill_rows(r, carry):
            rows_v[r, pl.ds(0, 16)] = one16
            return carry
        lax.fori_loop(0, _CHUNK, fill_rows, 0)

        plsc.subcore_barrier()

        def group(g, carry):
            gbase = wid * _CPT + g * _IDXG
            pltpu.sync_copy(dst_hbm.at[pl.ds(gbase, _IDXG)], dst_v)
            sd = {}
            for j in range(_IDXG):
                if j >= 8:
                    sd[j - 8].wait()
                sd[j] = pltpu.async_copy(rows_v, agg_sh.at[dst_v.at[j]],
                                         dsems[j % 8], add=True)
            for j in range(_IDXG - 8, _IDXG):
                sd[j].wait()
            return carry
        lax.fori_loop(0, _CPT // _IDXG, group, 0)

        plsc.subcore_barrier()

        obase = c * _NPAD + base
        pltpu.sync_copy(agg_sh.at[pl.ds(base, _RPT)], deg_out.at[pl.ds(obase, _RPT)])

    return pl.kernel(body, out_type=out_type, mesh=mesh, scratch_types=scratch)


def _make_sc_agg():
    mesh = plsc.VectorSubcoreMesh(core_axis_name="c", subcore_axis_name="s",
                                  num_cores=_NC, num_subcores=_NS)
    out_type = [jax.ShapeDtypeStruct((_NC * _NPAD, _D), jnp.float32)]
    scratch = [
        pltpu.VMEM((_IDXG, _CHUNK), jnp.int32),
        pltpu.VMEM((_IDXG, _CHUNK), jnp.int32),
        pltpu.VMEM((_CHUNK, _D), jnp.float32),
        pltpu.VMEM((_CHUNK, _D), jnp.float32),
        pltpu.VMEM_SHARED((_NPAD, _D), jnp.float32),
        pltpu.SemaphoreType.DMA,
        pltpu.SemaphoreType.DMA,
        pltpu.SemaphoreType.DMA,
        pltpu.SemaphoreType.DMA,
    ]

    _ZCHUNKS = (128, 128, 128, 128, _RPT - 512)

    def body(h_hbm, src_hbm, dst_hbm, agg_out, src_v, dst_v, rows_a, rows_b,
             agg_sh, sem_a, sem_b, ssem_a, ssem_b):
        c = lax.axis_index("c")
        s = lax.axis_index("s")
        wid = c * _NS + s
        zero16 = jnp.zeros((16,), jnp.float32)

        def zero_rows(r, carry):
            for kk in range(_D // 16):
                rows_a[r, pl.ds(kk * 16, 16)] = zero16
            return carry
        lax.fori_loop(0, _CHUNK, zero_rows, 0)

        base = s * _RPT
        off = 0
        for nn in _ZCHUNKS:
            pltpu.sync_copy(rows_a.at[pl.ds(0, nn)], agg_sh.at[pl.ds(base + off, nn)])
            off += nn

        plsc.subcore_barrier()

        bufs = (rows_a, rows_b)
        sems = (sem_a, sem_b)
        ssems = (ssem_a, ssem_b)
        _NB = 2

        def group(g, carry):
            gbase = wid * _CPT + g * _IDXG
            pltpu.sync_copy(src_hbm.at[pl.ds(gbase, _IDXG)], src_v)
            pltpu.sync_copy(dst_hbm.at[pl.ds(gbase, _IDXG)], dst_v)
            gd = {0: pltpu.async_copy(h_hbm.at[src_v.at[0]], bufs[0], sems[0])}
            sd = {}
            for j in range(_IDXG):
                if j + 1 < _IDXG:
                    if j >= _NB - 1:
                        sd[j - (_NB - 1)].wait()
                    gd[j + 1] = pltpu.async_copy(
                        h_hbm.at[src_v.at[j + 1]], bufs[(j + 1) % _NB],
                        sems[(j + 1) % _NB])
                gd[j].wait()
                sd[j] = pltpu.async_copy(
                    bufs[j % _NB], agg_sh.at[dst_v.at[j]], ssems[j % _NB], add=True)
            for j in range(_IDXG - _NB, _IDXG):
                sd[j].wait()
            return carry
        lax.fori_loop(0, _CPT // _IDXG, group, 0)

        plsc.subcore_barrier()

        obase = c * _NPAD + base
        pltpu.sync_copy(agg_sh.at[pl.ds(base, _RPT)], agg_out.at[pl.ds(obase, _RPT)])

    return pl.kernel(body, out_type=out_type, mesh=mesh, scratch_types=scratch)


_RL = 1024
_GL = 10

def _tc_layer(h, a0, a1, d0, d1, wsT, wnT, b):
    def body(h_r, a0_r, a1_r, d0_r, d1_r, ws_r, wn_r, b_r, o_r):
        deg = jnp.maximum(d0_r[:, 0:1] + d1_r[:, 0:1], 1.0)
        mean = (a0_r[...] + a1_r[...]) / deg
        acc = jnp.dot(h_r[...], ws_r[...], preferred_element_type=jnp.float32)
        acc = acc + jnp.dot(mean, wn_r[...], preferred_element_type=jnp.float32)
        o_r[...] = jnp.maximum(acc + b_r[...], 0.0)

    bs_rows = pl.BlockSpec((_RL, _D), lambda i: (i, 0))
    bs_d = pl.BlockSpec((_RL, 16), lambda i: (i, 0))
    bs_w = pl.BlockSpec((_D, _D), lambda i: (0, 0))
    bs_b = pl.BlockSpec((1, _D), lambda i: (0, 0))
    return pl.pallas_call(
        body,
        grid=(_GL,),
        in_specs=[bs_rows, bs_rows, bs_rows, bs_d, bs_d, bs_w, bs_w, bs_b],
        out_specs=bs_rows,
        out_shape=jax.ShapeDtypeStruct((_N, _D), jnp.float32),
    )(h, a0, a1, d0, d1, wsT, wnT, b)


_RH = 1024
_GH = 10
_NH = _RH * _GH


def _bn_relu(z, g, b):
    m = jnp.mean(z, axis=0, keepdims=True)
    zc = z - m
    v = jnp.mean(zc * zc, axis=0, keepdims=True)
    return jnp.maximum(zc / jnp.sqrt(v + _EPS) * g + b, 0.0)


def _tc_head(hp, bt3, adme_p, w0p, w0a, hb0, g0, b0, w1t, hb1, g1, b1,
             w2t, hb2, g2, b2, w3t, hb3):
    def body(h_r, bt_r, adme_r, w0p_r, w0a_r, hb0_r, g0_r, b0_r,
             w1_r, hb1_r, g1_r, b1_r, w2_r, hb2_r, g2_r, b2_r,
             w3_r, hb3_r, o_r, pool_acc, cnt_acc):
        i = pl.program_id(0)

        @pl.when(i == 0)
        def _():
            pool_acc[...] = jnp.zeros_like(pool_acc)
            cnt_acc[...] = jnp.zeros_like(cnt_acc)

        bt = bt_r[...].reshape(1, _RH)
        ids = lax.broadcasted_iota(jnp.int32, (_B, _RH), 0)
        mask = (bt == ids).astype(jnp.float32)
        pool_acc[...] += jnp.dot(mask, h_r[...], preferred_element_type=jnp.float32)
        cnt_acc[...] += jnp.broadcast_to(
            jnp.sum(mask, axis=1, keepdims=True), (_B, _D))

        @pl.when(i == _GH - 1)
        def _():
            pooled = pool_acc[...] / jnp.maximum(cnt_acc[...], 1.0)
            z = jnp.dot(pooled, w0p_r[...], preferred_element_type=jnp.float32)
            z = z + jnp.dot(adme_r[...], w0a_r[...], preferred_element_type=jnp.float32)
            z = _bn_relu(z + hb0_r[...], g0_r[...], b0_r[...])
            z = jnp.dot(z, w1_r[...], preferred_element_type=jnp.float32) + hb1_r[...]
            z = _bn_relu(z, g1_r[...], b1_r[...])
            z = jnp.dot(z, w2_r[...], preferred_element_type=jnp.float32) + hb2_r[...]
            z = _bn_relu(z, g2_r[...], b2_r[...])
            o_r[...] = jnp.dot(z, w3_r[...], preferred_element_type=jnp.float32) + hb3_r[...]

    def full(shape):
        return pl.BlockSpec(shape, lambda i: tuple(0 for _ in shape))

    return pl.pallas_call(
        body,
        grid=(_GH,),
        in_specs=[
            pl.BlockSpec((_RH, _D), lambda i: (i, 0)),
            pl.BlockSpec((1, 1, _RH), lambda i: (i, 0, 0)),
            full((_B, 16)),
            full((_D, 256)), full((16, 256)), full((1, 256)), full((1, 256)), full((1, 256)),
            full((256, _D)), full((1, _D)), full((1, _D)), full((1, _D)),
            full((_D, 64)), full((1, 64)), full((1, 64)), full((1, 64)),
            full((64, 1)), full((1, 1)),
        ],
        out_specs=pl.BlockSpec((_B, 1), lambda i: (0, 0)),
        out_shape=jax.ShapeDtypeStruct((_B, 1), jnp.float32),
        scratch_shapes=[
            pltpu.VMEM((_B, _D), jnp.float32),
            pltpu.VMEM((_B, _D), jnp.float32),
        ],
    )(hp, bt3, adme_p, w0p, w0a, hb0, g0, b0, w1t, hb1, g1, b1,
      w2t, hb2, g2, b2, w3t, hb3)


def kernel(x, edge_index, batch, adme_features, sage_Wself, sage_Wneigh, sage_b,
           hW0, hb0, g0, b0, hW1, hb1, g1, b1, hW2, hb2, g2, b2, hW3, hb3):
    pad = _E_PAD - _E
    src_p = jnp.concatenate(
        [edge_index[0], jnp.zeros((pad,), jnp.int32)]).reshape(_TOT_CHUNKS, _CHUNK)
    dst_p = jnp.concatenate(
        [edge_index[1], jnp.full((pad,), _N, jnp.int32)]).reshape(_TOT_CHUNKS, _CHUNK)

    agg_fn = _make_sc_agg()

    def run_agg(tab):
        r = agg_fn(tab, src_p, dst_p)
        if isinstance(r, (list, tuple)):
            r = r[0]
        return r

    deg_flat = _make_sc_deg()(dst_p)
    if isinstance(deg_flat, (list, tuple)):
        deg_flat = deg_flat[0]
    d0 = deg_flat[:_N, :16]
    d1 = deg_flat[_NPAD:_NPAD + _N, :16]

    h = x
    for i in range(_L):
        agg_flat = run_agg(h)
        a0 = agg_flat[:_N]
        a1 = agg_flat[_NPAD:_NPAD + _N]
        h = _tc_layer(h, a0, a1, d0, d1, sage_Wself[i].T, sage_Wneigh[i].T,
                      sage_b[i].reshape(1, _D))

    hp = jnp.concatenate([h, jnp.zeros((_NH - _N, _D), jnp.float32)], axis=0)
    bt3 = jnp.concatenate(
        [batch, jnp.full((_NH - _N,), _B, jnp.int32)]).reshape(_GH, 1, _RH)
    adme_p = jnp.pad(adme_features, ((0, 0), (0, 1)))
    w0p = hW0[:, :_D].T
    w0a = jnp.pad(hW0[:, _D:].T, ((0, 1), (0, 0)))

    out = _tc_head(hp, bt3, adme_p, w0p, w0a,
                   hb0.reshape(1, -1), g0.reshape(1, -1), b0.reshape(1, -1),
                   hW1.T, hb1.reshape(1, -1), g1.reshape(1, -1), b1.reshape(1, -1),
                   hW2.T, hb2.reshape(1, -1), g2.reshape(1, -1), b2.reshape(1, -1),
                   hW3.T, hb3.reshape(1, 1))
    return out.reshape(_B)


# speedup vs baseline: 3.0757x; 1.0002x over previous
import jax
import jax.numpy as jnp
from jax import lax
from jax.experimental import pallas as pl
from jax.experimental.pallas import tpu as pltpu
from jax.experimental.pallas import tpu_sc as plsc

_N = 10000
_E = 320000
_D = 128
_B = 64
_L = 5
_EPS = 1e-5

_NC = 2
_NS = 16
_TILES = _NC * _NS
_CHUNK = 128
_CPT = 80
_TOT_CHUNKS = _CPT * _TILES
_E_PAD = _TOT_CHUNKS * _CHUNK
_NPAD = 10112
_RPT = _NPAD // _NS
_IDXG = 16


def _make_sc_deg():
    mesh = plsc.VectorSubcoreMesh(core_axis_name="c", subcore_axis_name="s",
                                  num_cores=_NC, num_subcores=_NS)
    out_type = [jax.ShapeDtypeStruct((_NC * _NPAD, _D), jnp.float32)]
    scratch = [
        pltpu.VMEM((_IDXG, _CHUNK), jnp.int32),
        pltpu.VMEM((_CHUNK, _D), jnp.float32),
        pltpu.VMEM_SHARED((_NPAD, _D), jnp.float32),
    ] + [pltpu.SemaphoreType.DMA] * 8

    _ZCHUNKS = (128, 128, 128, 128, _RPT - 512)

    def body(dst_hbm, deg_out, dst_v, rows_v, agg_sh, *dsems):
        c = lax.axis_index("c")
        s = lax.axis_index("s")
        wid = c * _NS + s
        zero16 = jnp.zeros((16,), jnp.float32)

        def zero_rows(r, carry):
            for kk in range(_D // 16):
                rows_v[r, pl.ds(kk * 16, 16)] = zero16
            return carry
        lax.fori_loop(0, _CHUNK, zero_rows, 0)

        base = s * _RPT
        off = 0
        for nn in _ZCHUNKS:
            pltpu.sync_copy(rows_v.at[pl.ds(0, nn)], agg_sh.at[pl.ds(base + off, nn)])
            off += nn

        one16 = jnp.ones((16,), jnp.float32)

        def fill_rows(r, carry):
            rows_v[r, pl.ds(0, 16)] = one16
            return carry
        lax.fori_loop(0, _CHUNK, fill_rows, 0)

        plsc.subcore_barrier()

        def group(g, carry):
            gbase = wid * _CPT + g * _IDXG
            pltpu.sync_copy(dst_hbm.at[pl.ds(gbase, _IDXG)], dst_v)
            sd = {}
            for j in range(_IDXG):
                if j >= 8:
                    sd[j - 8].wait()
                sd[j] = pltpu.async_copy(rows_v, agg_sh.at[dst_v.at[j]],
                                         dsems[j % 8], add=True)
            for j in range(_IDXG - 8, _IDXG):
                sd[j].wait()
            return carry
        lax.fori_loop(0, _CPT // _IDXG, group, 0)

        plsc.subcore_barrier()

        obase = c * _NPAD + base
        pltpu.sync_copy(agg_sh.at[pl.ds(base, _RPT)], deg_out.at[pl.ds(obase, _RPT)])

    return pl.kernel(body, out_type=out_type, mesh=mesh, scratch_types=scratch)


def _make_sc_agg():
    mesh = plsc.VectorSubcoreMesh(core_axis_name="c", subcore_axis_name="s",
                                  num_cores=_NC, num_subcores=_NS)
    out_type = [jax.ShapeDtypeStruct((_NC * _NPAD, _D), jnp.float32)]
    scratch = [
        pltpu.VMEM((_IDXG, _CHUNK), jnp.int32),
        pltpu.VMEM((_IDXG, _CHUNK), jnp.int32),
        pltpu.VMEM((_CHUNK, _D), jnp.float32),
        pltpu.VMEM((_CHUNK, _D), jnp.float32),
        pltpu.VMEM_SHARED((_NPAD, _D), jnp.float32),
        pltpu.SemaphoreType.DMA,
        pltpu.SemaphoreType.DMA,
        pltpu.SemaphoreType.DMA,
        pltpu.SemaphoreType.DMA,
    ]

    _ZCHUNKS = (128, 128, 128, 128, _RPT - 512)

    def body(h_hbm, src_hbm, dst_hbm, agg_out, src_v, dst_v, rows_a, rows_b,
             agg_sh, sem_a, sem_b, ssem_a, ssem_b):
        c = lax.axis_index("c")
        s = lax.axis_index("s")
        wid = c * _NS + s
        zero16 = jnp.zeros((16,), jnp.float32)

        def zero_rows(r, carry):
            for kk in range(_D // 16):
                rows_a[r, pl.ds(kk * 16, 16)] = zero16
            return carry
        lax.fori_loop(0, _CHUNK, zero_rows, 0)

        base = s * _RPT
        off = 0
        for nn in _ZCHUNKS:
            pltpu.sync_copy(rows_a.at[pl.ds(0, nn)], agg_sh.at[pl.ds(base + off, nn)])
            off += nn

        plsc.subcore_barrier()

        bufs = (rows_a, rows_b)
        sems = (sem_a, sem_b)
        ssems = (ssem_a, ssem_b)
        _NB = 2

        def group(g, carry):
            gbase = wid * _CPT + g * _IDXG
            pltpu.sync_copy(src_hbm.at[pl.ds(gbase, _IDXG)], src_v)
            pltpu.sync_copy(dst_hbm.at[pl.ds(gbase, _IDXG)], dst_v)
            gd = {0: pltpu.async_copy(h_hbm.at[src_v.at[0]], bufs[0], sems[0])}
            sd = {}
            for j in range(_IDXG):
                if j + 1 < _IDXG:
                    if j >= _NB - 1:
                        sd[j - (_NB - 1)].wait()
                    gd[j + 1] = pltpu.async_copy(
                        h_hbm.at[src_v.at[j + 1]], bufs[(j + 1) % _NB],
                        sems[(j + 1) % _NB])
                gd[j].wait()
                sd[j] = pltpu.async_copy(
                    bufs[j % _NB], agg_sh.at[dst_v.at[j]], ssems[j % _NB], add=True)
            for j in range(_IDXG - _NB, _IDXG):
                sd[j].wait()
            return carry
        lax.fori_loop(0, _CPT // _IDXG, group, 0)

        plsc.subcore_barrier()

        obase = c * _NPAD + base
        pltpu.sync_copy(agg_sh.at[pl.ds(base, _RPT)], agg_out.at[pl.ds(obase, _RPT)])

    return pl.kernel(body, out_type=out_type, mesh=mesh, scratch_types=scratch)


_RL = 1024
_GL = 10

def _tc_layer(h, a0, a1, d0, d1, wsT, wnT, b):
    def body(h_r, a0_r, a1_r, d0_r, d1_r, ws_r, wn_r, b_r, o_r):
        deg = jnp.maximum(d0_r[:, 0:1] + d1_r[:, 0:1], 1.0)
        mean = (a0_r[...] + a1_r[...]) / deg
        acc = jnp.dot(h_r[...], ws_r[...], preferred_element_type=jnp.float32)
        acc = acc + jnp.dot(mean, wn_r[...], preferred_element_type=jnp.float32)
        o_r[...] = jnp.maximum(acc + b_r[...], 0.0)

    bs_rows = pl.BlockSpec((_RL, _D), lambda i: (i, 0))
    bs_d = pl.BlockSpec((_RL, 16), lambda i: (i, 0))
    bs_w = pl.BlockSpec((_D, _D), lambda i: (0, 0))
    bs_b = pl.BlockSpec((1, _D), lambda i: (0, 0))
    return pl.pallas_call(
        body,
        grid=(_GL,),
        in_specs=[bs_rows, bs_rows, bs_rows, bs_d, bs_d, bs_w, bs_w, bs_b],
        out_specs=bs_rows,
        out_shape=jax.ShapeDtypeStruct((_N, _D), jnp.float32),
    )(h, a0, a1, d0, d1, wsT, wnT, b)


_RH = 1024
_GH = 10
_NH = _RH * _GH


def _bn_relu(z, g, b):
    m = jnp.mean(z, axis=0, keepdims=True)
    zc = z - m
    v = jnp.mean(zc * zc, axis=0, keepdims=True)
    return jnp.maximum(zc / jnp.sqrt(v + _EPS) * g + b, 0.0)


def _tc_head(hp, bt3, adme_p, w0p, w0a, hb0, g0, b0, w1t, hb1, g1, b1,
             w2t, hb2, g2, b2, w3t, hb3):
    def body(h_r, bt_r, adme_r, w0p_r, w0a_r, hb0_r, g0_r, b0_r,
             w1_r, hb1_r, g1_r, b1_r, w2_r, hb2_r, g2_r, b2_r,
             w3_r, hb3_r, o_r, pool_acc, cnt_acc):
        i = pl.program_id(0)

        @pl.when(i == 0)
        def _():
            pool_acc[...] = jnp.zeros_like(pool_acc)
            cnt_acc[...] = jnp.zeros_like(cnt_acc)

        bt = bt_r[...].reshape(1, _RH)
        ids = lax.broadcasted_iota(jnp.int32, (_B, _RH), 0)
        mask = (bt == ids).astype(jnp.float32)
        pool_acc[...] += jnp.dot(mask, h_r[...], preferred_element_type=jnp.float32)
        cnt_acc[...] += jnp.broadcast_to(
            jnp.sum(mask, axis=1, keepdims=True), (_B, _D))

        @pl.when(i == _GH - 1)
        def _():
            pooled = pool_acc[...] / jnp.maximum(cnt_acc[...], 1.0)
            z = jnp.dot(pooled, w0p_r[...], preferred_element_type=jnp.float32)
            z = z + jnp.dot(adme_r[...], w0a_r[...], preferred_element_type=jnp.float32)
            z = _bn_relu(z + hb0_r[...], g0_r[...], b0_r[...])
            z = jnp.dot(z, w1_r[...], preferred_element_type=jnp.float32) + hb1_r[...]
            z = _bn_relu(z, g1_r[...], b1_r[...])
            z = jnp.dot(z, w2_r[...], preferred_element_type=jnp.float32) + hb2_r[...]
            z = _bn_relu(z, g2_r[...], b2_r[...])
            o_r[...] = jnp.dot(z, w3_r[...], preferred_element_type=jnp.float32) + hb3_r[...]

    def full(shape):
        return pl.BlockSpec(shape, lambda i: tuple(0 for _ in shape))

    return pl.pallas_call(
        body,
        grid=(_GH,),
        in_specs=[
            pl.BlockSpec((_RH, _D), lambda i: (i, 0)),
            pl.BlockSpec((1, 1, _RH), lambda i: (i, 0, 0)),
            full((_B, 16)),
            full((_D, 256)), full((16, 256)), full((1, 256)), full((1, 256)), full((1, 256)),
            full((256, _D)), full((1, _D)), full((1, _D)), full((1, _D)),
            full((_D, 64)), full((1, 64)), full((1, 64)), full((1, 64)),
            full((64, 1)), full((1, 1)),
        ],
        out_specs=pl.BlockSpec((_B, 1), lambda i: (0, 0)),
        out_shape=jax.ShapeDtypeStruct((_B, 1), jnp.float32),
        scratch_shapes=[
            pltpu.VMEM((_B, _D), jnp.float32),
            pltpu.VMEM((_B, _D), jnp.float32),
        ],
    )(hp, bt3, adme_p, w0p, w0a, hb0, g0, b0, w1t, hb1, g1, b1,
      w2t, hb2, g2, b2, w3t, hb3)


def kernel(x, edge_index, batch, adme_features, sage_Wself, sage_Wneigh, sage_b,
           hW0, hb0, g0, b0, hW1, hb1, g1, b1, hW2, hb2, g2, b2, hW3, hb3):
    pad = _E_PAD - _E
    src_p = jnp.concatenate(
        [edge_index[0], jnp.zeros((pad,), jnp.int32)]).reshape(_TOT_CHUNKS, _CHUNK)
    dst_p = jnp.concatenate(
        [edge_index[1], jnp.full((pad,), _N, jnp.int32)]).reshape(_TOT_CHUNKS, _CHUNK)

    agg_fn = _make_sc_agg()

    def run_agg(tab):
        r = agg_fn(tab, src_p, dst_p)
        if isinstance(r, (list, tuple)):
            r = r[0]
        return r

    deg_flat = _make_sc_deg()(dst_p)
    if isinstance(deg_flat, (list, tuple)):
        deg_flat = deg_flat[0]
    d0 = deg_flat[:_N, :16]
    d1 = deg_flat[_NPAD:_NPAD + _N, :16]

    h = x
    for i in range(_L):
        agg_flat = run_agg(h)
        a0 = agg_flat[:_N]
        a1 = agg_flat[_NPAD:_NPAD + _N]
        h = _tc_layer(h, a0, a1, d0, d1, sage_Wself[i].T, sage_Wneigh[i].T,
                      sage_b[i].reshape(1, _D))

    hp = jnp.concatenate([h, jnp.zeros((_NH - _N, _D), jnp.float32)], axis=0)
    bt3 = jnp.concatenate(
        [batch, jnp.full((_NH - _N,), _B, jnp.int32)]).reshape(_GH, 1, _RH)
    adme_p = jnp.pad(adme_features, ((0, 0), (0, 1)))
    w0p = hW0[:, :_D].T
    w0a = jnp.pad(hW0[:, _D:].T, ((0, 1), (0, 0)))

    out = _tc_head(hp, bt3, adme_p, w0p, w0a,
                   hb0.reshape(1, -1), g0.reshape(1, -1), b0.reshape(1, -1),
                   hW1.T, hb1.reshape(1, -1), g1.reshape(1, -1), b1.reshape(1, -1),
                   hW2.T, hb2.reshape(1, -1), g2.reshape(1, -1), b2.reshape(1, -1),
                   hW3.T, hb3.reshape(1, 1))
    return out.reshape(_B)
